# baseline (speedup 1.0000x reference)
import functools

import jax
import jax.numpy as jnp
from jax import lax
from jax.experimental import pallas as pl
from jax.experimental.pallas import tpu as pltpu
from jax.experimental.pallas import tpu_sc as plsc

NC = 2
NS = 16
NW = NC * NS
LANES = 16
CH = 32

SEQ_BLK = 512
TC_BATCH = 3


def _sc_body(in_hbm, w_hbm, out_hbm, w_v, x_v):
    wid = lax.axis_index("s") * NC + lax.axis_index("c")
    batch, seq_len, dim = in_hbm.shape
    rows_per_w = seq_len // NW
    s_base = wid * rows_per_w
    vecs_per_row = dim // LANES

    def jloop(j, _):
        s0 = s_base + j * CH
        pltpu.sync_copy(w_hbm.at[pl.ds(s0, CH), :], w_v)
        pltpu.sync_copy(in_hbm.at[batch - 1, pl.ds(s0, CH), :], x_v)

        def rloop(r, _):
            for c in range(vecs_per_row):
                sl = pl.ds(c * LANES, LANES)
                x_v[r, sl] = x_v[r, sl] + w_v[r, sl]
            return 0

        lax.fori_loop(0, CH, rloop, 0)
        pltpu.sync_copy(x_v, out_hbm.at[0, pl.ds(s0, CH), :])
        return 0

    lax.fori_loop(0, rows_per_w // CH, jloop, 0)


def _tc_add(x_ref, w_ref, o_ref):
    o_ref[...] = x_ref[...] + w_ref[...][None, :, :]


def kernel(inputs, W):
    batch, seq_len, dim = inputs.shape

    mesh = plsc.VectorSubcoreMesh(core_axis_name="c", subcore_axis_name="s")
    sc_call = functools.partial(
        pl.kernel,
        out_type=jax.ShapeDtypeStruct((1, seq_len, dim), inputs.dtype),
        mesh=mesh,
        scratch_types=[
            pltpu.VMEM((CH, dim), jnp.float32),
            pltpu.VMEM((CH, dim), jnp.float32),
        ],
    )(_sc_body)
    sc_out = sc_call(inputs, W)

    tc_out = pl.pallas_call(
        _tc_add,
        grid=(seq_len // SEQ_BLK,),
        in_specs=[
            pl.BlockSpec((TC_BATCH, SEQ_BLK, dim), lambda i: (0, i, 0)),
            pl.BlockSpec((SEQ_BLK, dim), lambda i: (i, 0)),
        ],
        out_specs=pl.BlockSpec((TC_BATCH, SEQ_BLK, dim), lambda i: (0, i, 0)),
        out_shape=jax.ShapeDtypeStruct((batch, seq_len, dim), inputs.dtype),
    )(inputs, W)

    return jnp.concatenate([tc_out[:TC_BATCH], sc_out], axis=0)


# speedup vs baseline: 1.8172x; 1.8172x over previous
import functools

import jax
import jax.numpy as jnp
from jax import lax
from jax.experimental import pallas as pl
from jax.experimental.pallas import tpu as pltpu
from jax.experimental.pallas import tpu_sc as plsc

NC = 2
NS = 16
NW = NC * NS
LANES = 16
CH = 16

SEQ_BLK = 512
TC_BATCH = 3


def _sc_body(in_hbm, w_hbm, out_hbm, w0, w1, x0, x1,
             sw0, sw1, sx0, sx1, so0, so1):
    wid = lax.axis_index("s") * NC + lax.axis_index("c")
    batch, seq_len, dim = in_hbm.shape
    rows_w = seq_len // NW
    nstages = rows_w // CH
    s_base = wid * rows_w
    b_last = batch - 1
    vecs = dim // LANES

    def w_copy(j, wb, sw):
        return pltpu.make_async_copy(
            w_hbm.at[pl.ds(s_base + j * CH, CH), :], wb, sw)

    def x_copy(j, xb, sx):
        return pltpu.make_async_copy(
            in_hbm.at[b_last, pl.ds(s_base + j * CH, CH), :], xb, sx)

    def o_copy(j, xb, so):
        return pltpu.make_async_copy(
            xb, out_hbm.at[0, pl.ds(s_base + j * CH, CH), :], so)

    def compute(xb, wb):
        def rloop(r, _):
            for c in range(vecs):
                sl = pl.ds(c * LANES, LANES)
                xb[r, sl] = xb[r, sl] + wb[r, sl]
            return 0
        lax.fori_loop(0, CH, rloop, 0)

    w_copy(0, w0, sw0).start()
    x_copy(0, x0, sx0).start()

    def stage(j, wb, sw, xb, sx, so, wn, swn, xn, sxn, son, first):
        if not first:
            o_copy(j - 1, xn, son).wait()

        @pl.when(j + 1 < nstages)
        def _():
            w_copy(j + 1, wn, swn).start()
            x_copy(j + 1, xn, sxn).start()

        w_copy(j, wb, sw).wait()
        x_copy(j, xb, sx).wait()
        compute(xb, wb)
        o_copy(j, xb, so).start()

    def loop(j2, _):
        j = 2 * j2

        @pl.when(j2 == 0)
        def _():
            stage(j, w0, sw0, x0, sx0, so0, w1, sw1, x1, sx1, so1, True)

        @pl.when(j2 > 0)
        def _():
            stage(j, w0, sw0, x0, sx0, so0, w1, sw1, x1, sx1, so1, False)

        stage(j + 1, w1, sw1, x1, sx1, so1, w0, sw0, x0, sx0, so0, False)
        return 0

    lax.fori_loop(0, nstages // 2, loop, 0)
    o_copy(nstages - 1, x1, so1).wait()


def _tc_add(x_ref, w_ref, o_ref):
    o_ref[...] = x_ref[...] + w_ref[...][None, :, :]


def _merge(tc_ref, sc_ref, o_ref):
    o_ref[...] = sc_ref[...]


def kernel(inputs, W):
    batch, seq_len, dim = inputs.shape

    mesh = plsc.VectorSubcoreMesh(core_axis_name="c", subcore_axis_name="s")
    sc_call = functools.partial(
        pl.kernel,
        out_type=jax.ShapeDtypeStruct((1, seq_len, dim), inputs.dtype),
        mesh=mesh,
        scratch_types=[
            pltpu.VMEM((CH, dim), jnp.float32),
            pltpu.VMEM((CH, dim), jnp.float32),
            pltpu.VMEM((CH, dim), jnp.float32),
            pltpu.VMEM((CH, dim), jnp.float32),
            pltpu.SemaphoreType.DMA,
            pltpu.SemaphoreType.DMA,
            pltpu.SemaphoreType.DMA,
            pltpu.SemaphoreType.DMA,
            pltpu.SemaphoreType.DMA,
            pltpu.SemaphoreType.DMA,
        ],
    )(_sc_body)
    sc_out = sc_call(inputs, W)

    tc_out = pl.pallas_call(
        _tc_add,
        grid=(seq_len // SEQ_BLK,),
        in_specs=[
            pl.BlockSpec((TC_BATCH, SEQ_BLK, dim), lambda i: (0, i, 0)),
            pl.BlockSpec((SEQ_BLK, dim), lambda i: (i, 0)),
        ],
        out_specs=pl.BlockSpec((TC_BATCH, SEQ_BLK, dim), lambda i: (0, i, 0)),
        out_shape=jax.ShapeDtypeStruct((batch, seq_len, dim), inputs.dtype),
    )(inputs, W)

    return pl.pallas_call(
        _merge,
        grid=(seq_len // SEQ_BLK,),
        in_specs=[
            pl.BlockSpec(memory_space=pl.ANY),
            pl.BlockSpec((1, SEQ_BLK, dim), lambda i: (0, i, 0)),
        ],
        out_specs=pl.BlockSpec((1, SEQ_BLK, dim), lambda i: (TC_BATCH, i, 0)),
        out_shape=jax.ShapeDtypeStruct((batch, seq_len, dim), inputs.dtype),
        input_output_aliases={0: 0},
    )(tc_out, sc_out)


# speedup vs baseline: 2.8507x; 1.5687x over previous
import jax
import jax.numpy as jnp
from jax.experimental import pallas as pl


SEQ_BLK = 512


def _add_kernel(x_ref, w_ref, o_ref):
    o_ref[...] = x_ref[...] + w_ref[...][None, :, :]


def kernel(inputs, W):
    batch, seq_len, dim = inputs.shape
    grid = (seq_len // SEQ_BLK,)
    return pl.pallas_call(
        _add_kernel,
        grid=grid,
        in_specs=[
            pl.BlockSpec((batch, SEQ_BLK, dim), lambda i: (0, i, 0)),
            pl.BlockSpec((SEQ_BLK, dim), lambda i: (i, 0)),
        ],
        out_specs=pl.BlockSpec((batch, SEQ_BLK, dim), lambda i: (0, i, 0)),
        out_shape=jax.ShapeDtypeStruct((batch, seq_len, dim), inputs.dtype),
    )(inputs, W)
